# Initial kernel scaffold; baseline (speedup 1.0000x reference)
import functools
import math

import jax
import jax.numpy as jnp
from jax import lax
from jax.experimental import pallas as pl
from jax.experimental.pallas import tpu as pltpu
from jax.experimental.pallas import tpu_sc as plsc

OUT_X = OUT_Y = OUT_Z = 12
NVOX = OUT_X * OUT_Y * OUT_Z
CAP = 63
N_ROIS = 128
N_PTS = 16384
C = 16
L = 16
NW = 32
ROIS_PER_W = N_ROIS // NW
N_CHUNKS = N_PTS // L
GCHUNK = 128
LIST_LEN = N_PTS + 2 * GCHUNK
NPRM = 12


def _sc_body(xs_hbm, ys_hbm, zs_hbm, prm_hbm, feat_hbm, out_hbm,
             xs_v, ys_v, zs_v, prm_v, grid_v, cnt_v, vox_v, pid_v,
             rows_v, sem):
    cid = lax.axis_index("c")
    sid = lax.axis_index("s")
    wid = sid * 2 + cid

    pltpu.sync_copy(xs_hbm, xs_v)
    pltpu.sync_copy(ys_hbm, ys_v)
    pltpu.sync_copy(zs_hbm, zs_v)
    pltpu.sync_copy(prm_hbm, prm_v)

    lane = lax.iota(jnp.int32, L)
    zeros_i = jnp.zeros((L,), jnp.int32)
    zeros_f = jnp.zeros((L,), jnp.float32)
    neg_inf = jnp.full((L,), -jnp.inf, jnp.float32)

    def per_roi(r, _):
        roi = wid * ROIS_PER_W + r
        pb = roi * NPRM
        cx = prm_v[pb + 0]
        cy = prm_v[pb + 1]
        czb = prm_v[pb + 2]
        cz = prm_v[pb + 3]
        hl = prm_v[pb + 4]
        hw = prm_v[pb + 5]
        hh = prm_v[pb + 6]
        cosa = prm_v[pb + 7]
        sina = prm_v[pb + 8]
        xr = prm_v[pb + 9]
        yr = prm_v[pb + 10]
        zr = prm_v[pb + 11]

        def init_grid(v, _):
            grid_v[pl.ds(v * L, L)] = neg_inf
            return 0
        lax.fori_loop(0, NVOX, init_grid, 0)

        def init_cnt(k, _):
            cnt_v[pl.ds(k * L, L)] = zeros_i
            return 0
        lax.fori_loop(0, NVOX // L, init_cnt, 0)

        def chunk(i, off):
            base = i * L
            px = xs_v[pl.ds(base, L)]
            py = ys_v[pl.ds(base, L)]
            pz = zs_v[pl.ds(base, L)]
            dx = px - cx
            dy = py - cy
            local_x = dx * cosa + dy * (-sina)
            local_y = dx * sina + dy * cosa
            local_z = pz - czb
            inside = ((jnp.abs(pz - cz) <= hh)
                      & (local_x > -hl) & (local_x < hl)
                      & (local_y > -hw) & (local_y < hw))
            x_idx = jnp.clip(((local_x + hl) / xr).astype(jnp.int32),
                             0, OUT_X - 1)
            y_idx = jnp.clip(((local_y + hw) / yr).astype(jnp.int32),
                             0, OUT_Y - 1)
            z_idx = jnp.clip((local_z / zr).astype(jnp.int32),
                             0, OUT_Z - 1)
            vox = (x_idx * OUT_Y + y_idx) * OUT_Z + z_idx
            pid = base + lane
            plsc.store_compressed(vox_v.at[pl.ds(off, L)], vox, mask=inside)
            plsc.store_compressed(pid_v.at[pl.ds(off, L)], pid, mask=inside)
            return off + jnp.sum(inside.astype(jnp.int32))
        n = lax.fori_loop(0, N_CHUNKS, chunk, jnp.int32(0))

        def padz(k, _):
            pid_v[pl.ds(n + k * L, L)] = zeros_i
            return 0
        lax.fori_loop(0, GCHUNK // L, padz, 0)

        nch = (n + (GCHUNK - 1)) // GCHUNK

        def bchunk(ci, _):
            gbase = ci * GCHUNK
            pltpu.async_copy(
                feat_hbm.at[pid_v.at[pl.ds(gbase, GCHUNK)]], rows_v, sem
            ).wait()
            m = jnp.minimum(GCHUNK, n - gbase)

            def scat(j, _):
                v = vox_v[gbase + j]
                c0 = cnt_v[v]
                g = grid_v[pl.ds(v * L, L)]
                frow = rows_v[j]
                take = c0 < CAP
                grid_v[pl.ds(v * L, L)] = jnp.where(
                    take, jnp.maximum(g, frow), g)
                cnt_v[v] = jnp.where(take, c0 + 1, c0)
                return 0
            lax.fori_loop(0, m, scat, 0)
            return 0
        lax.fori_loop(0, nch, bchunk, 0)

        def fix(v, _):
            g = grid_v[pl.ds(v * L, L)]
            grid_v[pl.ds(v * L, L)] = jnp.where(g == neg_inf, zeros_f, g)
            return 0
        lax.fori_loop(0, NVOX, fix, 0)

        pltpu.sync_copy(grid_v, out_hbm.at[roi])
        return 0

    lax.fori_loop(0, ROIS_PER_W, per_roi, 0)


_mesh = plsc.VectorSubcoreMesh(core_axis_name="c", subcore_axis_name="s")

_pooled_call = functools.partial(
    pl.kernel,
    out_type=jax.ShapeDtypeStruct((N_ROIS, NVOX * C), jnp.float32),
    mesh=_mesh,
    scratch_types=[
        pltpu.VMEM((N_PTS,), jnp.float32),
        pltpu.VMEM((N_PTS,), jnp.float32),
        pltpu.VMEM((N_PTS,), jnp.float32),
        pltpu.VMEM((N_ROIS * NPRM,), jnp.float32),
        pltpu.VMEM((NVOX * C,), jnp.float32),
        pltpu.VMEM((NVOX,), jnp.int32),
        pltpu.VMEM((LIST_LEN,), jnp.int32),
        pltpu.VMEM((LIST_LEN,), jnp.int32),
        pltpu.VMEM((GCHUNK, C), jnp.float32),
        pltpu.SemaphoreType.DMA,
    ],
)(_sc_body)


def kernel(rois, pts, pts_feature):
    cx = rois[:, 0]
    cy = rois[:, 1]
    czb = rois[:, 2]
    w = rois[:, 3]
    l = rois[:, 4]
    h = rois[:, 5]
    rz = rois[:, 6]
    cz = czb + h * 0.5
    rot = rz + jnp.pi * 0.5
    prm = jnp.stack(
        [cx, cy, czb, cz, l * 0.5, w * 0.5, h * 0.5,
         jnp.cos(rot), jnp.sin(rot),
         l / OUT_X, w / OUT_Y, h / OUT_Z],
        axis=1,
    ).reshape(-1).astype(jnp.float32)
    xs = jnp.asarray(pts[:, 0], jnp.float32)
    ys = jnp.asarray(pts[:, 1], jnp.float32)
    zs = jnp.asarray(pts[:, 2], jnp.float32)
    out = _pooled_call(xs, ys, zs, prm, pts_feature)
    return out.reshape(N_ROIS, OUT_X, OUT_Y, OUT_Z, C)


# speedup vs baseline: 67.1671x; 67.1671x over previous
import functools
import math

import jax
import jax.numpy as jnp
from jax import lax
from jax.experimental import pallas as pl
from jax.experimental.pallas import tpu as pltpu
from jax.experimental.pallas import tpu_sc as plsc

OUT_X = OUT_Y = OUT_Z = 12
NVOX = OUT_X * OUT_Y * OUT_Z
CAP = 63
N_ROIS = 128
N_PTS = 16384
C = 16
L = 16
NW = 32
ROIS_PER_W = N_ROIS // NW
N_CHUNKS = N_PTS // L
GCHUNK = 128
LIST_LEN = N_PTS + 2 * GCHUNK
NPRM = 16


def _sc_body(xs_hbm, ys_hbm, zs_hbm, prm_hbm, feat_hbm, out_hbm,
             xs_v, ys_v, zs_v, prm_v, grid_v, cnt_v, vox_v, pid_v,
             rows_v, sem):
    cid = lax.axis_index("c")
    sid = lax.axis_index("s")
    wid = sid * 2 + cid

    pltpu.sync_copy(xs_hbm, xs_v)
    pltpu.sync_copy(ys_hbm, ys_v)
    pltpu.sync_copy(zs_hbm, zs_v)
    pltpu.sync_copy(prm_hbm, prm_v)

    lane = lax.iota(jnp.int32, L)
    zeros_i = jnp.zeros((L,), jnp.int32)
    zeros_f = jnp.zeros((L,), jnp.float32)
    neg_inf = jnp.full((L,), -jnp.inf, jnp.float32)

    def per_roi(r, _):
        roi = wid * ROIS_PER_W + r
        pvec = prm_v[pl.ds(roi * NPRM, L)]
        cx = pvec[0]
        cy = pvec[1]
        czb = pvec[2]
        cz = pvec[3]
        hl = pvec[4]
        hw = pvec[5]
        hh = pvec[6]
        cosa = pvec[7]
        sina = pvec[8]
        xr = pvec[9]
        yr = pvec[10]
        zr = pvec[11]

        def init_grid(v, _):
            grid_v[pl.ds(v * L, L)] = neg_inf
            return 0
        lax.fori_loop(0, NVOX, init_grid, 0)

        def init_cnt(k, _):
            cnt_v[pl.ds(k * L, L)] = zeros_i
            return 0
        lax.fori_loop(0, NVOX // L + 1, init_cnt, 0)

        def chunk(i, off):
            base = i * L
            px = xs_v[pl.ds(base, L)]
            py = ys_v[pl.ds(base, L)]
            pz = zs_v[pl.ds(base, L)]
            dx = px - cx
            dy = py - cy
            local_x = dx * cosa + dy * (-sina)
            local_y = dx * sina + dy * cosa
            local_z = pz - czb
            inside = ((jnp.abs(pz - cz) <= hh)
                      & (local_x > -hl) & (local_x < hl)
                      & (local_y > -hw) & (local_y < hw))
            x_idx = jnp.clip(((local_x + hl) / xr).astype(jnp.int32),
                             0, OUT_X - 1)
            y_idx = jnp.clip(((local_y + hw) / yr).astype(jnp.int32),
                             0, OUT_Y - 1)
            z_idx = jnp.clip((local_z / zr).astype(jnp.int32),
                             0, OUT_Z - 1)
            vox = (x_idx * OUT_Y + y_idx) * OUT_Z + z_idx
            pid = base + lane
            m_i = inside.astype(jnp.int32)
            inc = plsc.cumsum(m_i)
            dst = off + (inc - m_i)
            plsc.store_scatter(vox_v, [dst], vox, mask=inside)
            plsc.store_scatter(pid_v, [dst], pid, mask=inside)
            return off + inc[L - 1]
        n = lax.fori_loop(0, N_CHUNKS, chunk, jnp.int32(0))

        def padz(k, _):
            pid_v[pl.ds(n + k * L, L)] = zeros_i
            return 0
        lax.fori_loop(0, GCHUNK // L, padz, 0)

        nch = (n + (GCHUNK - 1)) // GCHUNK

        def bchunk(ci, _):
            gbase = ci * GCHUNK
            pltpu.async_copy(
                feat_hbm.at[pid_v.at[pl.ds(gbase, GCHUNK)]], rows_v, sem
            ).wait()
            m = jnp.minimum(GCHUNK, n - gbase)

            def scat(j, _):
                v = vox_v[pl.ds(gbase + j, L)][0]
                cvec = cnt_v[pl.ds(v, L)]
                c0 = cvec[0]
                g = grid_v[pl.ds(v * L, L)]
                frow = rows_v[j]
                take = c0 < CAP
                grid_v[pl.ds(v * L, L)] = jnp.where(
                    take, jnp.maximum(g, frow), g)
                cnt_v[pl.ds(v, L)] = jnp.where(
                    take & (lane == 0), cvec + 1, cvec)
                return 0
            lax.fori_loop(0, m, scat, 0)
            return 0
        lax.fori_loop(0, nch, bchunk, 0)

        def fix(v, _):
            g = grid_v[pl.ds(v * L, L)]
            grid_v[pl.ds(v * L, L)] = jnp.where(g == neg_inf, zeros_f, g)
            return 0
        lax.fori_loop(0, NVOX, fix, 0)

        pltpu.sync_copy(grid_v, out_hbm.at[roi])
        return 0

    lax.fori_loop(0, ROIS_PER_W, per_roi, 0)


_mesh = plsc.VectorSubcoreMesh(core_axis_name="c", subcore_axis_name="s")

_pooled_call = functools.partial(
    pl.kernel,
    out_type=jax.ShapeDtypeStruct((N_ROIS, NVOX * C), jnp.float32),
    mesh=_mesh,
    compiler_params=pltpu.CompilerParams(needs_layout_passes=False, use_tc_tiling_on_sc=False),
    scratch_types=[
        pltpu.VMEM((N_PTS,), jnp.float32),
        pltpu.VMEM((N_PTS,), jnp.float32),
        pltpu.VMEM((N_PTS,), jnp.float32),
        pltpu.VMEM((N_ROIS * NPRM,), jnp.float32),
        pltpu.VMEM((NVOX * C,), jnp.float32),
        pltpu.VMEM((NVOX + L,), jnp.int32),
        pltpu.VMEM((LIST_LEN,), jnp.int32),
        pltpu.VMEM((LIST_LEN,), jnp.int32),
        pltpu.VMEM((GCHUNK, C), jnp.float32),
        pltpu.SemaphoreType.DMA,
    ],
)(_sc_body)


def kernel(rois, pts, pts_feature):
    cx = rois[:, 0]
    cy = rois[:, 1]
    czb = rois[:, 2]
    w = rois[:, 3]
    l = rois[:, 4]
    h = rois[:, 5]
    rz = rois[:, 6]
    cz = czb + h * 0.5
    rot = rz + jnp.pi * 0.5
    zpad = jnp.zeros_like(cx)
    prm = jnp.stack(
        [cx, cy, czb, cz, l * 0.5, w * 0.5, h * 0.5,
         jnp.cos(rot), jnp.sin(rot),
         l / OUT_X, w / OUT_Y, h / OUT_Z,
         zpad, zpad, zpad, zpad],
        axis=1,
    ).reshape(-1).astype(jnp.float32)
    xs = jnp.asarray(pts[:, 0], jnp.float32)
    ys = jnp.asarray(pts[:, 1], jnp.float32)
    zs = jnp.asarray(pts[:, 2], jnp.float32)
    out = _pooled_call(xs, ys, zs, prm, pts_feature)
    return out.reshape(N_ROIS, OUT_X, OUT_Y, OUT_Z, C)
